# Initial kernel scaffold; baseline (speedup 1.0000x reference)
import jax
import jax.numpy as jnp
from jax.experimental import pallas as pl


def kernel(atom_feat, degree, segment_id, glob_feat, glob_mask, glob_valid_mask, seg_feat, seg_feat_mask, seg_valid_mask, token_feat, atom_table, degree_table, vnode, glob_W, glob_b, seg_W, seg_b):
    raise NotImplementedError("write your pallas kernel here")


# speedup vs baseline: 10.5376x; 10.5376x over previous
import jax
import jax.numpy as jnp
from jax import lax
from jax.experimental import pallas as pl
from jax.experimental.pallas import tpu as pltpu

_B, _N, _F, _D, _S = 256, 64, 9, 768, 16
_NG, _NS = 2, 2
_NA, _ND = 512, 512
_K = _NA + _ND
_BB = 8
_R = _BB * _N


def _body(ids_ref, tok_ref, gf_ref, gm_ref, gv_ref, sf_ref, sm_ref, sv_ref,
          vn_ref, gW_ref, gb_ref, sW_ref, sb_ref, tab_ref, out_ref):
    ids = ids_ref[...]
    iota = lax.broadcasted_iota(jnp.int32, (_R, _K), 1)
    cnt = jnp.zeros((_R, _K), jnp.bfloat16)
    for k in range(_F + 1):
        cnt += (ids[:, k:k + 1] == iota).astype(jnp.bfloat16)
    acc = jnp.dot(cnt, tab_ref[...], preferred_element_type=jnp.float32)
    atom = acc + tok_ref[...]

    cls = jnp.broadcast_to(vn_ref[...], (_BB, _D))

    gf = gf_ref[...]
    gm = gm_ref[...]
    gv = gv_ref[...]
    gW = gW_ref[...]
    gb = gb_ref[...]
    glob = (gm[:, 0:1] * (gf[:, 0:1] * gW[0:1, :] + gb[0:1, :])
            + gm[:, 1:2] * (gf[:, 1:2] * gW[1:2, :] + gb[1:2, :]))
    glob = glob * gv

    sf = sf_ref[...]
    sm = sm_ref[...]
    sv = sv_ref[...]
    sW = sW_ref[...]
    sb = sb_ref[...]
    a0 = sv * sm[:, 0:1]
    a1 = sv * sm[:, 1:2]
    seg = ((a0 * sf[:, 0:1]) * sW[0:1, :] + a0 * sb[0:1, :]
           + (a1 * sf[:, 1:2]) * sW[1:2, :] + a1 * sb[1:2, :])

    out_ref[:, 0:1, :] = cls.reshape(_BB, 1, _D)
    out_ref[:, 1:2, :] = glob.reshape(_BB, 1, _D)
    out_ref[:, 2:2 + _S, :] = seg.reshape(_BB, _S, _D)
    out_ref[:, 2 + _S:, :] = atom.reshape(_BB, _N, _D)


def kernel(atom_feat, degree, segment_id, glob_feat, glob_mask, glob_valid_mask,
           seg_feat, seg_feat_mask, seg_valid_mask, token_feat, atom_table,
           degree_table, vnode, glob_W, glob_b, seg_W, seg_b):
    del segment_id
    dtype = token_feat.dtype
    ids = jnp.concatenate(
        [atom_feat.reshape(_B * _N, _F), degree.reshape(_B * _N, 1) + _NA],
        axis=1)
    tok = token_feat.reshape(_B * _N, _D)
    sf = seg_feat.reshape(_B * _S, _NS)
    sm = seg_feat_mask.reshape(_B * _S, _NS)
    sv = seg_valid_mask.reshape(_B * _S, 1)
    tab = jnp.concatenate([atom_table, degree_table], axis=0).astype(jnp.bfloat16)

    grid = (_B // _BB,)
    out = pl.pallas_call(
        _body,
        grid=grid,
        in_specs=[
            pl.BlockSpec((_R, _F + 1), lambda i: (i, 0)),
            pl.BlockSpec((_R, _D), lambda i: (i, 0)),
            pl.BlockSpec((_BB, _NG), lambda i: (i, 0)),
            pl.BlockSpec((_BB, _NG), lambda i: (i, 0)),
            pl.BlockSpec((_BB, 1), lambda i: (i, 0)),
            pl.BlockSpec((_BB * _S, _NS), lambda i: (i, 0)),
            pl.BlockSpec((_BB * _S, _NS), lambda i: (i, 0)),
            pl.BlockSpec((_BB * _S, 1), lambda i: (i, 0)),
            pl.BlockSpec((1, _D), lambda i: (0, 0)),
            pl.BlockSpec((_NG, _D), lambda i: (0, 0)),
            pl.BlockSpec((_NG, _D), lambda i: (0, 0)),
            pl.BlockSpec((_NS, _D), lambda i: (0, 0)),
            pl.BlockSpec((_NS, _D), lambda i: (0, 0)),
            pl.BlockSpec((_K, _D), lambda i: (0, 0)),
        ],
        out_specs=pl.BlockSpec((_BB, 2 + _S + _N, _D), lambda i: (i, 0, 0)),
        out_shape=jax.ShapeDtypeStruct((_B, 2 + _S + _N, _D), dtype),
        compiler_params=pltpu.CompilerParams(
            dimension_semantics=("arbitrary",),
        ),
    )(ids, tok, glob_feat, glob_mask, glob_valid_mask, sf, sm, sv,
      vnode, glob_W, glob_b, seg_W, seg_b, tab)
    return out


# speedup vs baseline: 12.9660x; 1.2304x over previous
import jax
import jax.numpy as jnp
from jax import lax
from jax.experimental import pallas as pl
from jax.experimental.pallas import tpu as pltpu

_B, _N, _F, _D, _S = 256, 64, 9, 768, 16
_NG, _NS = 2, 2
_NA, _ND = 512, 512
_K = _NA + _ND
_BB = 8
_R = _BB * _N


def _body(ids_ref, tok_ref, gf_ref, gm_ref, gv_ref, sf_ref, sm_ref, sv_ref,
          vn_ref, gW_ref, gb_ref, sW_ref, sb_ref, atab_ref, dtab_ref, out_ref):
    ids = ids_ref[...]
    iota = lax.broadcasted_iota(jnp.int16, (_R, _NA), 1)
    cnt_i = (ids[:, 0:1] == iota).astype(jnp.int16)
    for k in range(1, _F):
        cnt_i += (ids[:, k:k + 1] == iota).astype(jnp.int16)
    cnt = cnt_i.astype(jnp.bfloat16)
    dcnt = (ids[:, _F:_F + 1] == iota).astype(jnp.bfloat16)
    acc = jnp.dot(cnt, atab_ref[...], preferred_element_type=jnp.float32)
    acc += jnp.dot(dcnt, dtab_ref[...], preferred_element_type=jnp.float32)
    atom = acc + tok_ref[...]

    cls = jnp.broadcast_to(vn_ref[...], (_BB, _D))

    gf = gf_ref[...]
    gm = gm_ref[...]
    gv = gv_ref[...]
    gW = gW_ref[...]
    gb = gb_ref[...]
    glob = (gm[:, 0:1] * (gf[:, 0:1] * gW[0:1, :] + gb[0:1, :])
            + gm[:, 1:2] * (gf[:, 1:2] * gW[1:2, :] + gb[1:2, :]))
    glob = glob * gv

    sf = sf_ref[...]
    sm = sm_ref[...]
    sv = sv_ref[...]
    sW = sW_ref[...]
    sb = sb_ref[...]
    a0 = sv * sm[:, 0:1]
    a1 = sv * sm[:, 1:2]
    seg = ((a0 * sf[:, 0:1]) * sW[0:1, :] + a0 * sb[0:1, :]
           + (a1 * sf[:, 1:2]) * sW[1:2, :] + a1 * sb[1:2, :])

    out_ref[:, 0:1, :] = cls.reshape(_BB, 1, _D)
    out_ref[:, 1:2, :] = glob.reshape(_BB, 1, _D)
    out_ref[:, 2:2 + _S, :] = seg.reshape(_BB, _S, _D)
    out_ref[:, 2 + _S:, :] = atom.reshape(_BB, _N, _D)


def kernel(atom_feat, degree, segment_id, glob_feat, glob_mask, glob_valid_mask,
           seg_feat, seg_feat_mask, seg_valid_mask, token_feat, atom_table,
           degree_table, vnode, glob_W, glob_b, seg_W, seg_b):
    del segment_id
    dtype = token_feat.dtype
    ids = jnp.concatenate(
        [atom_feat.reshape(_B * _N, _F), degree.reshape(_B * _N, 1)],
        axis=1).astype(jnp.int16)
    tok = token_feat.reshape(_B * _N, _D)
    sf = seg_feat.reshape(_B * _S, _NS)
    sm = seg_feat_mask.reshape(_B * _S, _NS)
    sv = seg_valid_mask.reshape(_B * _S, 1)
    atab = atom_table.astype(jnp.bfloat16)
    dtab = degree_table.astype(jnp.bfloat16)

    grid = (_B // _BB,)
    out = pl.pallas_call(
        _body,
        grid=grid,
        in_specs=[
            pl.BlockSpec((_R, _F + 1), lambda i: (i, 0)),
            pl.BlockSpec((_R, _D), lambda i: (i, 0)),
            pl.BlockSpec((_BB, _NG), lambda i: (i, 0)),
            pl.BlockSpec((_BB, _NG), lambda i: (i, 0)),
            pl.BlockSpec((_BB, 1), lambda i: (i, 0)),
            pl.BlockSpec((_BB * _S, _NS), lambda i: (i, 0)),
            pl.BlockSpec((_BB * _S, _NS), lambda i: (i, 0)),
            pl.BlockSpec((_BB * _S, 1), lambda i: (i, 0)),
            pl.BlockSpec((1, _D), lambda i: (0, 0)),
            pl.BlockSpec((_NG, _D), lambda i: (0, 0)),
            pl.BlockSpec((_NG, _D), lambda i: (0, 0)),
            pl.BlockSpec((_NS, _D), lambda i: (0, 0)),
            pl.BlockSpec((_NS, _D), lambda i: (0, 0)),
            pl.BlockSpec((_NA, _D), lambda i: (0, 0)),
            pl.BlockSpec((_ND, _D), lambda i: (0, 0)),
        ],
        out_specs=pl.BlockSpec((_BB, 2 + _S + _N, _D), lambda i: (i, 0, 0)),
        out_shape=jax.ShapeDtypeStruct((_B, 2 + _S + _N, _D), dtype),
        compiler_params=pltpu.CompilerParams(
            dimension_semantics=("arbitrary",),
        ),
    )(ids, tok, glob_feat, glob_mask, glob_valid_mask, sf, sm, sv,
      vnode, glob_W, glob_b, seg_W, seg_b, atab, dtab)
    return out


# speedup vs baseline: 13.7694x; 1.0620x over previous
import jax
import jax.numpy as jnp
from jax import lax
from jax.experimental import pallas as pl
from jax.experimental.pallas import tpu as pltpu

_B, _N, _F, _D, _S = 256, 64, 9, 768, 16
_NG, _NS = 2, 2
_NA, _ND = 512, 512
_BB = 16
_R = _BB * _N


def _body(ids_ref, tok_ref, gf_ref, gm_ref, gv_ref, sf_ref, sm_ref, sv_ref,
          vn_ref, gW_ref, gb_ref, sW_ref, sb_ref, atab_ref, dtab_ref, out_ref):
    ids = ids_ref[...]
    iota = lax.broadcasted_iota(jnp.int16, (_R, _NA), 1)
    cnt_i = (ids[:, 0:1] == iota).astype(jnp.int16)
    for k in range(1, _F):
        cnt_i += (ids[:, k:k + 1] == iota).astype(jnp.int16)
    cnt = cnt_i.astype(jnp.bfloat16)
    dcnt = (ids[:, _F:_F + 1] == iota).astype(jnp.bfloat16)
    acc = jnp.dot(cnt, atab_ref[...], preferred_element_type=jnp.float32)
    acc += jnp.dot(dcnt, dtab_ref[...], preferred_element_type=jnp.float32)
    atom = acc + tok_ref[...]

    cls = jnp.broadcast_to(vn_ref[...], (_BB, _D))

    gf = gf_ref[...]
    gm = gm_ref[...]
    gv = gv_ref[...]
    gW = gW_ref[...]
    gb = gb_ref[...]
    glob = (gm[:, 0:1] * (gf[:, 0:1] * gW[0:1, :] + gb[0:1, :])
            + gm[:, 1:2] * (gf[:, 1:2] * gW[1:2, :] + gb[1:2, :]))
    glob = glob * gv

    sf = sf_ref[...]
    sm = sm_ref[...]
    sv = sv_ref[...]
    sW = sW_ref[...]
    sb = sb_ref[...]
    a0 = sv * sm[:, 0:1]
    a1 = sv * sm[:, 1:2]
    seg = ((a0 * sf[:, 0:1]) * sW[0:1, :] + a0 * sb[0:1, :]
           + (a1 * sf[:, 1:2]) * sW[1:2, :] + a1 * sb[1:2, :])

    out_ref[:, 0:1, :] = cls.reshape(_BB, 1, _D)
    out_ref[:, 1:2, :] = glob.reshape(_BB, 1, _D)
    out_ref[:, 2:2 + _S, :] = seg.reshape(_BB, _S, _D)
    out_ref[:, 2 + _S:, :] = atom.reshape(_BB, _N, _D)


def kernel(atom_feat, degree, segment_id, glob_feat, glob_mask, glob_valid_mask,
           seg_feat, seg_feat_mask, seg_valid_mask, token_feat, atom_table,
           degree_table, vnode, glob_W, glob_b, seg_W, seg_b):
    del segment_id
    dtype = token_feat.dtype
    ids = jnp.concatenate(
        [atom_feat.reshape(_B * _N, _F), degree.reshape(_B * _N, 1)],
        axis=1).astype(jnp.int16)
    tok = token_feat.reshape(_B * _N, _D)
    sf = seg_feat.reshape(_B * _S, _NS)
    sm = seg_feat_mask.reshape(_B * _S, _NS)
    sv = seg_valid_mask.reshape(_B * _S, 1)
    atab = atom_table.astype(jnp.bfloat16)
    dtab = degree_table.astype(jnp.bfloat16)

    grid = (_B // _BB,)
    out = pl.pallas_call(
        _body,
        grid=grid,
        in_specs=[
            pl.BlockSpec((_R, _F + 1), lambda i: (i, 0)),
            pl.BlockSpec((_R, _D), lambda i: (i, 0)),
            pl.BlockSpec((_BB, _NG), lambda i: (i, 0)),
            pl.BlockSpec((_BB, _NG), lambda i: (i, 0)),
            pl.BlockSpec((_BB, 1), lambda i: (i, 0)),
            pl.BlockSpec((_BB * _S, _NS), lambda i: (i, 0)),
            pl.BlockSpec((_BB * _S, _NS), lambda i: (i, 0)),
            pl.BlockSpec((_BB * _S, 1), lambda i: (i, 0)),
            pl.BlockSpec((1, _D), lambda i: (0, 0)),
            pl.BlockSpec((_NG, _D), lambda i: (0, 0)),
            pl.BlockSpec((_NG, _D), lambda i: (0, 0)),
            pl.BlockSpec((_NS, _D), lambda i: (0, 0)),
            pl.BlockSpec((_NS, _D), lambda i: (0, 0)),
            pl.BlockSpec((_NA, _D), lambda i: (0, 0)),
            pl.BlockSpec((_ND, _D), lambda i: (0, 0)),
        ],
        out_specs=pl.BlockSpec((_BB, 2 + _S + _N, _D), lambda i: (i, 0, 0)),
        out_shape=jax.ShapeDtypeStruct((_B, 2 + _S + _N, _D), dtype),
        compiler_params=pltpu.CompilerParams(
            dimension_semantics=("arbitrary",),
        ),
    )(ids, tok, glob_feat, glob_mask, glob_valid_mask, sf, sm, sv,
      vnode, glob_W, glob_b, seg_W, seg_b, atab, dtab)
    return out
